# baseline (speedup 1.0000x reference)
import functools

import jax
import jax.numpy as jnp
from jax import lax
from jax.experimental import pallas as pl
from jax.experimental.pallas import tpu as pltpu
from jax.experimental.pallas import tpu_sc as plsc

N_NODES = 10000
N_EDGES = 320000
IN_FEATS = 128
N_HIDDEN = 32
DIM_Z = 16

NUM_CORES = 2
NUM_SUBCORES = 16
NUM_TILES = NUM_CORES * NUM_SUBCORES
E_PER_TILE = N_EDGES // NUM_TILES
CHUNK = 80
NCHUNKS = E_PER_TILE // CHUNK
ROWS_PER_SUBCORE = N_NODES // NUM_SUBCORES
D = 32

_MESH = plsc.VectorSubcoreMesh(core_axis_name="c", subcore_axis_name="s")


def _seg_gather_sum(src2, dst2, h, zeros_stripe):
  def body(src2_hbm, dst2_hbm, h_hbm, zeros_hbm, out_hbm, sidx, didx,
           rows0, rows1, rows2, rows3, acc, hsh,
           gsem0, gsem1, gsem2, gsem3, ssem0, ssem1, ssem2, ssem3):
    cid = lax.axis_index("c")
    sid = lax.axis_index("s")
    wid = cid * NUM_SUBCORES + sid
    r0 = sid * ROWS_PER_SUBCORE
    pltpu.sync_copy(zeros_hbm, acc.at[pl.ds(r0, ROWS_PER_SUBCORE)])
    pltpu.sync_copy(h_hbm.at[sid], hsh.at[pl.ds(r0, ROWS_PER_SUBCORE)])
    pltpu.sync_copy(src2_hbm.at[wid], sidx)
    pltpu.sync_copy(dst2_hbm.at[wid], didx)
    plsc.subcore_barrier()

    rows = (rows0, rows1, rows2, rows3)
    gsem = (gsem0, gsem1, gsem2, gsem3)
    ssem = (ssem0, ssem1, ssem2, ssem3)

    def wait_gather(i, b):
      pltpu.make_async_copy(hsh.at[sidx.at[i]], rows[b], gsem[b]).wait()

    def issue_gather(i, b):
      pltpu.async_copy(hsh.at[sidx.at[i]], rows[b], gsem[b])

    def issue_scatter(i, b):
      pltpu.async_copy(rows[b], acc.at[didx.at[i]], ssem[b], add=True)

    def wait_scatter(i, b):
      pltpu.make_async_copy(rows[b], acc.at[didx.at[i]], ssem[b]).wait()

    for b in (0, 1, 2):
      issue_gather(b, b)
    wait_gather(0, 0)
    issue_scatter(0, 0)
    issue_gather(3, 3)

    def group(g, carry):
      for k in range(4):
        i = 4 * g + 1 + k
        b = (1 + k) % 4
        wait_gather(i, b)
        issue_scatter(i, b)
        wait_scatter(i - 1, (b + 3) % 4)
        issue_gather(i + 3, (b + 3) % 4)
      return carry

    lax.fori_loop(0, 30, group, 0)
    i = 121
    wait_gather(i, 1)
    issue_scatter(i, 1)
    wait_scatter(i - 1, 0)
    issue_gather(i + 3, 0)
    for i, b in ((122, 2), (123, 3), (124, 0)):
      wait_gather(i, b)
      issue_scatter(i, b)
      wait_scatter(i - 1, (b + 3) % 4)
    wait_scatter(NCHUNKS - 1, 0)

    plsc.subcore_barrier()
    pltpu.sync_copy(acc.at[pl.ds(r0, ROWS_PER_SUBCORE)], out_hbm.at[cid, sid])

  k = pl.kernel(
      body,
      out_type=jax.ShapeDtypeStruct(
          (NUM_CORES, NUM_SUBCORES, ROWS_PER_SUBCORE, D), jnp.float32),
      mesh=_MESH,
      scratch_types=[
          pltpu.VMEM((NCHUNKS, CHUNK), jnp.int32),
          pltpu.VMEM((NCHUNKS, CHUNK), jnp.int32),
          pltpu.VMEM((CHUNK, D), jnp.float32),
          pltpu.VMEM((CHUNK, D), jnp.float32),
          pltpu.VMEM((CHUNK, D), jnp.float32),
          pltpu.VMEM((CHUNK, D), jnp.float32),
          pltpu.VMEM_SHARED((N_NODES, D), jnp.float32),
          pltpu.VMEM_SHARED((N_NODES, D), jnp.float32),
          pltpu.SemaphoreType.DMA,
          pltpu.SemaphoreType.DMA,
          pltpu.SemaphoreType.DMA,
          pltpu.SemaphoreType.DMA,
          pltpu.SemaphoreType.DMA,
          pltpu.SemaphoreType.DMA,
          pltpu.SemaphoreType.DMA,
          pltpu.SemaphoreType.DMA,
      ],
      compiler_params=pltpu.CompilerParams(use_tc_tiling_on_sc=False),
  )
  h3 = h.reshape(NUM_SUBCORES, ROWS_PER_SUBCORE, D)
  return k(src2, dst2, h3, zeros_stripe).reshape(NUM_CORES, N_NODES, D)


def _seg_degree(dst2, zeros_stripe):
  def body(dst2_hbm, zeros_hbm, out_hbm, didx, rows, acc,
           ssem0, ssem1, ssem2, ssem3):
    cid = lax.axis_index("c")
    sid = lax.axis_index("s")
    wid = cid * NUM_SUBCORES + sid
    r0 = sid * ROWS_PER_SUBCORE
    pltpu.sync_copy(zeros_hbm, acc.at[pl.ds(r0, ROWS_PER_SUBCORE)])
    pltpu.sync_copy(dst2_hbm.at[wid], didx)

    def fill(i, carry):
      rows[i, pl.ds(0, 16)] = jnp.ones((16,), jnp.float32)
      rows[i, pl.ds(16, 16)] = jnp.ones((16,), jnp.float32)
      return carry

    lax.fori_loop(0, CHUNK, fill, 0)
    plsc.subcore_barrier()

    ssem = (ssem0, ssem1, ssem2, ssem3)

    def issue(i, b):
      pltpu.async_copy(rows, acc.at[didx.at[i]], ssem[b], add=True)

    def wait(i, b):
      pltpu.make_async_copy(rows, acc.at[didx.at[i]], ssem[b]).wait()

    for b in range(4):
      issue(b, b)

    def group(g, carry):
      for k in range(4):
        i = 4 * g + k
        wait(i - 4, k)
        issue(i, k)
      return carry

    lax.fori_loop(1, 31, group, 0)
    wait(120, 0)
    issue(124, 0)
    for i, b in ((121, 1), (122, 2), (123, 3), (124, 0)):
      wait(i, b)
    plsc.subcore_barrier()
    pltpu.sync_copy(acc.at[pl.ds(r0, ROWS_PER_SUBCORE)], out_hbm.at[cid, sid])

  k = pl.kernel(
      body,
      out_type=jax.ShapeDtypeStruct(
          (NUM_CORES, NUM_SUBCORES, ROWS_PER_SUBCORE, D), jnp.float32),
      mesh=_MESH,
      scratch_types=[
          pltpu.VMEM((NCHUNKS, CHUNK), jnp.int32),
          pltpu.VMEM((CHUNK, D), jnp.float32),
          pltpu.VMEM_SHARED((N_NODES, D), jnp.float32),
          pltpu.SemaphoreType.DMA,
          pltpu.SemaphoreType.DMA,
          pltpu.SemaphoreType.DMA,
          pltpu.SemaphoreType.DMA,
      ],
      compiler_params=pltpu.CompilerParams(use_tc_tiling_on_sc=False),
  )
  return k(dst2, zeros_stripe).reshape(NUM_CORES, N_NODES, D)


BR = 1000


def _norm_block(deg_block):
  deg = deg_block[0, :, 0:1] + deg_block[1, :, 0:1]
  return jnp.where(deg > 0, lax.rsqrt(deg), 0.0)


def _tc1(deg_part, features, W_base):
  def body(deg_ref, x_ref, w_ref, o_ref):
    norm = _norm_block(deg_ref[...])
    o_ref[...] = jnp.dot(x_ref[...], w_ref[...],
                         preferred_element_type=jnp.float32) * norm

  return pl.pallas_call(
      body,
      grid=(N_NODES // BR,),
      in_specs=[
          pl.BlockSpec((NUM_CORES, BR, D), lambda i: (0, i, 0)),
          pl.BlockSpec((BR, IN_FEATS), lambda i: (i, 0)),
          pl.BlockSpec((IN_FEATS, N_HIDDEN), lambda i: (0, 0)),
      ],
      out_specs=pl.BlockSpec((BR, N_HIDDEN), lambda i: (i, 0)),
      out_shape=jax.ShapeDtypeStruct((N_NODES, N_HIDDEN), jnp.float32),
  )(deg_part, features, W_base)


def _tc2(deg_part, hid_part, W_cat):
  def body(deg_ref, hp_ref, w_ref, o_ref):
    norm = _norm_block(deg_ref[...])
    hidden = (hp_ref[0] + hp_ref[1]) * norm
    o_ref[...] = jnp.dot(hidden, w_ref[...],
                         preferred_element_type=jnp.float32) * norm

  return pl.pallas_call(
      body,
      grid=(N_NODES // BR,),
      in_specs=[
          pl.BlockSpec((NUM_CORES, BR, D), lambda i: (0, i, 0)),
          pl.BlockSpec((NUM_CORES, BR, D), lambda i: (0, i, 0)),
          pl.BlockSpec((N_HIDDEN, 2 * DIM_Z), lambda i: (0, 0)),
      ],
      out_specs=pl.BlockSpec((BR, 2 * DIM_Z), lambda i: (i, 0)),
      out_shape=jax.ShapeDtypeStruct((N_NODES, 2 * DIM_Z), jnp.float32),
  )(deg_part, hid_part, W_cat)


def _tc3(deg_part, ml_part, noise):
  def body(deg_ref, mp_ref, n_ref, o_ref):
    norm = _norm_block(deg_ref[...])
    h = (mp_ref[0] + mp_ref[1]) * norm
    mean = jnp.maximum(h[:, :DIM_Z], 0.0)
    logstd = jnp.maximum(h[:, DIM_Z:], 0.0)
    o_ref[...] = n_ref[...] * jnp.exp(logstd) + mean

  return pl.pallas_call(
      body,
      grid=(N_NODES // BR,),
      in_specs=[
          pl.BlockSpec((NUM_CORES, BR, D), lambda i: (0, i, 0)),
          pl.BlockSpec((NUM_CORES, BR, D), lambda i: (0, i, 0)),
          pl.BlockSpec((BR, DIM_Z), lambda i: (i, 0)),
      ],
      out_specs=pl.BlockSpec((BR, DIM_Z), lambda i: (i, 0)),
      out_shape=jax.ShapeDtypeStruct((N_NODES, DIM_Z), jnp.float32),
  )(deg_part, ml_part, noise)


BR4 = 400


def _tc4(Z):
  def body(zi_ref, zjt_ref, o_ref):
    o_ref[...] = lax.dot_general(zi_ref[...], zjt_ref[...],
                                 (((1,), (0,)), ((), ())),
                                 preferred_element_type=jnp.float32)

  return pl.pallas_call(
      body,
      grid=(N_NODES // BR4,),
      in_specs=[
          pl.BlockSpec((BR4, DIM_Z), lambda i: (i, 0)),
          pl.BlockSpec((DIM_Z, N_NODES), lambda i: (0, 0)),
      ],
      out_specs=pl.BlockSpec((BR4, N_NODES), lambda i: (i, 0)),
      out_shape=jax.ShapeDtypeStruct((N_NODES, N_NODES), jnp.float32),
  )(Z, Z.T)


def kernel(features, edge_index, W_base, W_mean, W_logstd):
  src2 = edge_index[0].reshape(NUM_TILES, NCHUNKS, CHUNK)
  dst2 = edge_index[1].reshape(NUM_TILES, NCHUNKS, CHUNK)
  zeros_stripe = jnp.zeros((ROWS_PER_SUBCORE, D), jnp.float32)

  deg_part = _seg_degree(dst2, zeros_stripe)
  h1pre = _tc1(deg_part, features, W_base)
  hid_part = _seg_gather_sum(src2, dst2, h1pre, zeros_stripe)
  W_cat = jnp.concatenate([W_mean, W_logstd], axis=1)
  mlpre = _tc2(deg_part, hid_part, W_cat)
  ml_part = _seg_gather_sum(src2, dst2, mlpre, zeros_stripe)
  noise = jax.random.normal(jax.random.key(42), (N_NODES, DIM_Z), jnp.float32)
  Z = _tc3(deg_part, ml_part, noise)
  return _tc4(Z)


# speedup vs baseline: 1.0304x; 1.0304x over previous
import functools

import jax
import jax.numpy as jnp
from jax import lax
from jax.experimental import pallas as pl
from jax.experimental.pallas import tpu as pltpu
from jax.experimental.pallas import tpu_sc as plsc

N_NODES = 10000
N_EDGES = 320000
IN_FEATS = 128
N_HIDDEN = 32
DIM_Z = 16

NUM_CORES = 2
NUM_SUBCORES = 16
NUM_TILES = NUM_CORES * NUM_SUBCORES
E_PER_TILE = N_EDGES // NUM_TILES
CHUNK = 80
NCHUNKS = E_PER_TILE // CHUNK
ROWS_PER_SUBCORE = N_NODES // NUM_SUBCORES
D = 32

_MESH = plsc.VectorSubcoreMesh(core_axis_name="c", subcore_axis_name="s")


def _seg_gather_sum(src2, dst2, h, zeros_stripe):
  def body(src2_hbm, dst2_hbm, h_hbm, zeros_hbm, out_hbm, sidx, didx,
           rows0, rows1, rows2, rows3, acc, hsh,
           gsem0, gsem1, gsem2, gsem3, ssem0, ssem1, ssem2, ssem3):
    cid = lax.axis_index("c")
    sid = lax.axis_index("s")
    wid = cid * NUM_SUBCORES + sid
    r0 = sid * ROWS_PER_SUBCORE
    pltpu.sync_copy(zeros_hbm, acc.at[pl.ds(r0, ROWS_PER_SUBCORE)])
    pltpu.sync_copy(h_hbm.at[sid], hsh.at[pl.ds(r0, ROWS_PER_SUBCORE)])
    pltpu.sync_copy(src2_hbm.at[wid], sidx)
    pltpu.sync_copy(dst2_hbm.at[wid], didx)
    plsc.subcore_barrier()

    rows = (rows0, rows1, rows2, rows3)
    gsem = (gsem0, gsem1, gsem2, gsem3)
    ssem = (ssem0, ssem1, ssem2, ssem3)

    def wait_gather(i, b):
      pltpu.make_async_copy(hsh.at[sidx.at[i]], rows[b], gsem[b]).wait()

    def issue_gather(i, b):
      pltpu.async_copy(hsh.at[sidx.at[i]], rows[b], gsem[b])

    def issue_scatter(i, b):
      pltpu.async_copy(rows[b], acc.at[didx.at[i]], ssem[b], add=True)

    def wait_scatter(i, b):
      pltpu.make_async_copy(rows[b], acc.at[didx.at[i]], ssem[b]).wait()

    for b in (0, 1, 2):
      issue_gather(b, b)
    wait_gather(0, 0)
    issue_scatter(0, 0)
    issue_gather(3, 3)

    def group(g, carry):
      for k in range(4):
        i = 4 * g + 1 + k
        b = (1 + k) % 4
        wait_gather(i, b)
        issue_scatter(i, b)
        wait_scatter(i - 1, (b + 3) % 4)
        issue_gather(i + 3, (b + 3) % 4)
      return carry

    lax.fori_loop(0, 30, group, 0)
    i = 121
    wait_gather(i, 1)
    issue_scatter(i, 1)
    wait_scatter(i - 1, 0)
    issue_gather(i + 3, 0)
    for i, b in ((122, 2), (123, 3), (124, 0)):
      wait_gather(i, b)
      issue_scatter(i, b)
      wait_scatter(i - 1, (b + 3) % 4)
    wait_scatter(NCHUNKS - 1, 0)

    plsc.subcore_barrier()
    pltpu.sync_copy(acc.at[pl.ds(r0, ROWS_PER_SUBCORE)], out_hbm.at[cid, sid])

  k = pl.kernel(
      body,
      out_type=jax.ShapeDtypeStruct(
          (NUM_CORES, NUM_SUBCORES, ROWS_PER_SUBCORE, D), jnp.float32),
      mesh=_MESH,
      scratch_types=[
          pltpu.VMEM((NCHUNKS, CHUNK), jnp.int32),
          pltpu.VMEM((NCHUNKS, CHUNK), jnp.int32),
          pltpu.VMEM((CHUNK, D), jnp.float32),
          pltpu.VMEM((CHUNK, D), jnp.float32),
          pltpu.VMEM((CHUNK, D), jnp.float32),
          pltpu.VMEM((CHUNK, D), jnp.float32),
          pltpu.VMEM_SHARED((N_NODES, D), jnp.float32),
          pltpu.VMEM_SHARED((N_NODES, D), jnp.float32),
          pltpu.SemaphoreType.DMA,
          pltpu.SemaphoreType.DMA,
          pltpu.SemaphoreType.DMA,
          pltpu.SemaphoreType.DMA,
          pltpu.SemaphoreType.DMA,
          pltpu.SemaphoreType.DMA,
          pltpu.SemaphoreType.DMA,
          pltpu.SemaphoreType.DMA,
      ],
      compiler_params=pltpu.CompilerParams(use_tc_tiling_on_sc=False),
  )
  h3 = h.reshape(NUM_SUBCORES, ROWS_PER_SUBCORE, D)
  return k(src2, dst2, h3, zeros_stripe).reshape(NUM_CORES, N_NODES, D)


D_DEG = 16


def _seg_degree(dst2, zeros_stripe16):
  def body(dst2_hbm, zeros_hbm, out_hbm, didx, rows, acc,
           ssem0, ssem1, ssem2, ssem3):
    cid = lax.axis_index("c")
    sid = lax.axis_index("s")
    wid = cid * NUM_SUBCORES + sid
    r0 = sid * ROWS_PER_SUBCORE
    pltpu.sync_copy(zeros_hbm, acc.at[pl.ds(r0, ROWS_PER_SUBCORE)])
    pltpu.sync_copy(dst2_hbm.at[wid], didx)

    def fill(i, carry):
      rows[i, pl.ds(0, 16)] = jnp.ones((16,), jnp.float32)
      return carry

    lax.fori_loop(0, CHUNK, fill, 0)
    plsc.subcore_barrier()

    ssem = (ssem0, ssem1, ssem2, ssem3)

    def issue(i, b):
      pltpu.async_copy(rows, acc.at[didx.at[i]], ssem[b], add=True)

    def wait(i, b):
      pltpu.make_async_copy(rows, acc.at[didx.at[i]], ssem[b]).wait()

    for b in range(4):
      issue(b, b)

    def group(g, carry):
      for k in range(4):
        i = 4 * g + k
        wait(i - 4, k)
        issue(i, k)
      return carry

    lax.fori_loop(1, 31, group, 0)
    wait(120, 0)
    issue(124, 0)
    for i, b in ((121, 1), (122, 2), (123, 3), (124, 0)):
      wait(i, b)
    plsc.subcore_barrier()
    pltpu.sync_copy(acc.at[pl.ds(r0, ROWS_PER_SUBCORE)], out_hbm.at[cid, sid])

  k = pl.kernel(
      body,
      out_type=jax.ShapeDtypeStruct(
          (NUM_CORES, NUM_SUBCORES, ROWS_PER_SUBCORE, D_DEG), jnp.float32),
      mesh=_MESH,
      scratch_types=[
          pltpu.VMEM((NCHUNKS, CHUNK), jnp.int32),
          pltpu.VMEM((CHUNK, D_DEG), jnp.float32),
          pltpu.VMEM_SHARED((N_NODES, D_DEG), jnp.float32),
          pltpu.SemaphoreType.DMA,
          pltpu.SemaphoreType.DMA,
          pltpu.SemaphoreType.DMA,
          pltpu.SemaphoreType.DMA,
      ],
      compiler_params=pltpu.CompilerParams(use_tc_tiling_on_sc=False),
  )
  return k(dst2, zeros_stripe16).reshape(NUM_CORES, N_NODES, D_DEG)


BR = 1000


def _norm_block(deg_block):
  deg = deg_block[0, :, 0:1] + deg_block[1, :, 0:1]
  return jnp.where(deg > 0, lax.rsqrt(deg), 0.0)


def _tc1(deg_part, features, W_base):
  def body(deg_ref, x_ref, w_ref, o_ref):
    norm = _norm_block(deg_ref[...])
    o_ref[...] = jnp.dot(x_ref[...], w_ref[...],
                         preferred_element_type=jnp.float32) * norm

  return pl.pallas_call(
      body,
      grid=(N_NODES // BR,),
      in_specs=[
          pl.BlockSpec((NUM_CORES, BR, D_DEG), lambda i: (0, i, 0)),
          pl.BlockSpec((BR, IN_FEATS), lambda i: (i, 0)),
          pl.BlockSpec((IN_FEATS, N_HIDDEN), lambda i: (0, 0)),
      ],
      out_specs=pl.BlockSpec((BR, N_HIDDEN), lambda i: (i, 0)),
      out_shape=jax.ShapeDtypeStruct((N_NODES, N_HIDDEN), jnp.float32),
  )(deg_part, features, W_base)


def _tc2(deg_part, hid_part, W_cat):
  def body(deg_ref, hp_ref, w_ref, o_ref):
    norm = _norm_block(deg_ref[...])
    hidden = (hp_ref[0] + hp_ref[1]) * norm
    o_ref[...] = jnp.dot(hidden, w_ref[...],
                         preferred_element_type=jnp.float32) * norm

  return pl.pallas_call(
      body,
      grid=(N_NODES // BR,),
      in_specs=[
          pl.BlockSpec((NUM_CORES, BR, D_DEG), lambda i: (0, i, 0)),
          pl.BlockSpec((NUM_CORES, BR, D), lambda i: (0, i, 0)),
          pl.BlockSpec((N_HIDDEN, 2 * DIM_Z), lambda i: (0, 0)),
      ],
      out_specs=pl.BlockSpec((BR, 2 * DIM_Z), lambda i: (i, 0)),
      out_shape=jax.ShapeDtypeStruct((N_NODES, 2 * DIM_Z), jnp.float32),
  )(deg_part, hid_part, W_cat)


def _tc3(deg_part, ml_part, noise):
  def body(deg_ref, mp_ref, n_ref, o_ref):
    norm = _norm_block(deg_ref[...])
    h = (mp_ref[0] + mp_ref[1]) * norm
    mean = jnp.maximum(h[:, :DIM_Z], 0.0)
    logstd = jnp.maximum(h[:, DIM_Z:], 0.0)
    o_ref[...] = n_ref[...] * jnp.exp(logstd) + mean

  return pl.pallas_call(
      body,
      grid=(N_NODES // BR,),
      in_specs=[
          pl.BlockSpec((NUM_CORES, BR, D_DEG), lambda i: (0, i, 0)),
          pl.BlockSpec((NUM_CORES, BR, D), lambda i: (0, i, 0)),
          pl.BlockSpec((BR, DIM_Z), lambda i: (i, 0)),
      ],
      out_specs=pl.BlockSpec((BR, DIM_Z), lambda i: (i, 0)),
      out_shape=jax.ShapeDtypeStruct((N_NODES, DIM_Z), jnp.float32),
  )(deg_part, ml_part, noise)


BR4 = 400


def _tc4(Z):
  def body(zi_ref, zj_ref, o_ref):
    o_ref[...] = lax.dot_general(zi_ref[...], zj_ref[...],
                                 (((1,), (1,)), ((), ())),
                                 preferred_element_type=jnp.float32)

  return pl.pallas_call(
      body,
      grid=(N_NODES // BR4,),
      in_specs=[
          pl.BlockSpec((BR4, DIM_Z), lambda i: (i, 0)),
          pl.BlockSpec((N_NODES, DIM_Z), lambda i: (0, 0)),
      ],
      out_specs=pl.BlockSpec((BR4, N_NODES), lambda i: (i, 0)),
      out_shape=jax.ShapeDtypeStruct((N_NODES, N_NODES), jnp.float32),
  )(Z, Z)


def kernel(features, edge_index, W_base, W_mean, W_logstd):
  src2 = edge_index[0].reshape(NUM_TILES, NCHUNKS, CHUNK)
  dst2 = edge_index[1].reshape(NUM_TILES, NCHUNKS, CHUNK)
  zeros_stripe = jnp.zeros((ROWS_PER_SUBCORE, D), jnp.float32)
  zeros_stripe16 = jnp.zeros((ROWS_PER_SUBCORE, D_DEG), jnp.float32)

  deg_part = _seg_degree(dst2, zeros_stripe16)
  h1pre = _tc1(deg_part, features, W_base)
  hid_part = _seg_gather_sum(src2, dst2, h1pre, zeros_stripe)
  W_cat = jnp.concatenate([W_mean, W_logstd], axis=1)
  mlpre = _tc2(deg_part, hid_part, W_cat)
  ml_part = _seg_gather_sum(src2, dst2, mlpre, zeros_stripe)
  noise = jax.random.normal(jax.random.key(42), (N_NODES, DIM_Z), jnp.float32)
  Z = _tc3(deg_part, ml_part, noise)
  return _tc4(Z)
